# baseline (speedup 1.0000x reference)
import jax
import jax.numpy as jnp
from jax.experimental import pallas as pl
from jax.experimental.pallas import tpu as pltpu

_B = 16
_T = 512
_DE = 512
_DP = 320
_DJ = 320
_V = 1024
_BLANK = 0
_TCH = 128
_MB = 1024

_f32 = jnp.float32


def _proj_kernel(enc_ref, wenc_ref, bj_ref, out_ref):
    out_ref[...] = (jnp.dot(enc_ref[...], wenc_ref[...],
                            preferred_element_type=_f32)
                    + bj_ref[...])


def _gtab_kernel(embed_ref, wih_ref, bl_ref, out_ref):
    out_ref[...] = (jnp.dot(embed_ref[...], wih_ref[...],
                            preferred_element_type=_f32)
                    + bl_ref[...])


def _decode_kernel(encp_ref, lens_ref, gtab_ref,
                   whh_ref,
                   wpred_ref, wout_ref, bout_ref,
                   lab_ref, sc_ref):
    iota_v = jax.lax.broadcasted_iota(jnp.int32, (_B, _V), 1)
    iota_c = jax.lax.broadcasted_iota(jnp.int32, (_B, _TCH), 1)
    lens = lens_ref[...][:, :1]

    def step(chunk):
        def body(tt, carry):
            h, c, lbl, labacc, scacc = carry
            t = chunk * _TCH + tt

            onehot = (iota_v == lbl).astype(_f32)
            gates = (jnp.dot(onehot, gtab_ref[...],
                             preferred_element_type=_f32)
                     + jnp.dot(h, whh_ref[...], preferred_element_type=_f32))
            g_i = gates[:, 0:_DP]
            g_f = gates[:, _DP:2 * _DP]
            g_g = gates[:, 2 * _DP:3 * _DP]
            g_o = gates[:, 3 * _DP:4 * _DP]
            c_new = (jax.nn.sigmoid(g_f) * c
                     + jax.nn.sigmoid(g_i) * jnp.tanh(g_g))
            h_new = jax.nn.sigmoid(g_o) * jnp.tanh(c_new)

            pre = encp_ref[t] + jnp.dot(h_new, wpred_ref[...],
                                        preferred_element_type=_f32)
            logits = (jnp.dot(jnp.tanh(pre), wout_ref[...],
                              preferred_element_type=_f32)
                      + bout_ref[...])

            m = jnp.max(logits, axis=1, keepdims=True)
            sym = jnp.argmax(logits, axis=1, keepdims=True).astype(jnp.int32)
            score = -jnp.log(jnp.sum(jnp.exp(logits - m),
                                     axis=1, keepdims=True))

            blank = jnp.logical_or(sym == _BLANK, t >= lens)
            h = jnp.where(blank, h, h_new)
            c = jnp.where(blank, c, c_new)
            lbl = jnp.where(blank, lbl, sym)
            emit = jnp.where(blank, _BLANK, sym)

            colmask = iota_c == tt
            labacc = jnp.where(colmask,
                               jnp.broadcast_to(emit, (_B, _TCH)), labacc)
            scacc = jnp.where(colmask,
                              jnp.broadcast_to(score, (_B, _TCH)), scacc)
            return h, c, lbl, labacc, scacc
        return body

    h = jnp.zeros((_B, _DP), _f32)
    c = jnp.zeros((_B, _DP), _f32)
    lbl = jnp.full((_B, 1), _BLANK, jnp.int32)
    for chunk in range(_T // _TCH):
        init = (h, c, lbl,
                jnp.zeros((_B, _TCH), jnp.int32),
                jnp.zeros((_B, _TCH), _f32))
        h, c, lbl, labacc, scacc = jax.lax.fori_loop(
            0, _TCH, step(chunk), init, unroll=8)
        lab_ref[:, chunk * _TCH:(chunk + 1) * _TCH] = labacc
        sc_ref[:, chunk * _TCH:(chunk + 1) * _TCH] = scacc


def _full(shape):
    return pl.BlockSpec(shape, lambda i: (0,) * len(shape))


@jax.jit
def kernel(encoded_outs, encoded_lens, embed, W_ih, W_hh, b_lstm,
           W_enc, W_pred, b_joint, W_out, b_out):
    enc_flat = jnp.transpose(encoded_outs, (1, 0, 2)).reshape(_T * _B, _DE)

    encp = pl.pallas_call(
        _proj_kernel,
        grid=(_T * _B // _MB,),
        in_specs=[
            pl.BlockSpec((_MB, _DE), lambda i: (i, 0)),
            pl.BlockSpec((_DE, _DJ), lambda i: (0, 0)),
            pl.BlockSpec((1, _DJ), lambda i: (0, 0)),
        ],
        out_specs=pl.BlockSpec((_MB, _DJ), lambda i: (i, 0)),
        out_shape=jax.ShapeDtypeStruct((_T * _B, _DJ), _f32),
    )(enc_flat, W_enc, b_joint[None, :])
    encp = encp.reshape(_T, _B, _DJ)

    gtab = pl.pallas_call(
        _gtab_kernel,
        grid=(1,),
        in_specs=[
            _full((_V, _DP)),
            _full((_DP, 4 * _DP)),
            _full((1, 4 * _DP)),
        ],
        out_specs=_full((_V, 4 * _DP)),
        out_shape=jax.ShapeDtypeStruct((_V, 4 * _DP), _f32),
    )(embed, W_ih, b_lstm[None, :])

    lens_b = jnp.broadcast_to(encoded_lens.astype(jnp.int32)[:, None],
                              (_B, 128))

    labels, scores = pl.pallas_call(
        _decode_kernel,
        grid=(1,),
        in_specs=[
            _full((_T, _B, _DJ)),
            _full((_B, 128)),
            _full((_V, 4 * _DP)),
            _full((_DP, 4 * _DP)),
            _full((_DP, _DJ)),
            _full((_DJ, _V)),
            _full((1, _V)),
        ],
        out_specs=[
            _full((_B, _T)),
            _full((_B, _T)),
        ],
        out_shape=[
            jax.ShapeDtypeStruct((_B, _T), jnp.int32),
            jax.ShapeDtypeStruct((_B, _T), _f32),
        ],
        compiler_params=pltpu.CompilerParams(
            dimension_semantics=("arbitrary",)),
    )(encp, lens_b, gtab, W_hh,
      W_pred, W_out, b_out[None, :])
    return labels, scores


# speedup vs baseline: 1.1849x; 1.1849x over previous
import jax
import jax.numpy as jnp
from jax.experimental import pallas as pl
from jax.experimental.pallas import tpu as pltpu

_B = 16
_T = 512
_DE = 512
_DP = 320
_DJ = 320
_V = 1024
_BLANK = 0
_TCH = 128
_MB = 1024

_f32 = jnp.float32


def _proj_kernel(enc_ref, wenc_ref, bj_ref, out_ref):
    out_ref[...] = (jnp.dot(enc_ref[...], wenc_ref[...],
                            preferred_element_type=_f32)
                    + bj_ref[...])


def _decode_kernel(encp_ref, lens_ref, embed_ref,
                   wih_ref, whh_ref, bl_ref,
                   wpred_ref, wout_ref, bout_ref,
                   lab_ref, sc_ref):
    iota_v = jax.lax.broadcasted_iota(jnp.int32, (_B, _V), 1)
    iota_c = jax.lax.broadcasted_iota(jnp.int32, (_B, _TCH), 1)
    lens = lens_ref[...][:, :1]

    def step(chunk):
        def body(tt, carry):
            h, c, lbl, labacc, scacc = carry
            t = chunk * _TCH + tt

            onehot = (iota_v == lbl).astype(_f32)
            emb = jnp.dot(onehot, embed_ref[...],
                          preferred_element_type=_f32)

            gates = (jnp.dot(emb, wih_ref[...], preferred_element_type=_f32)
                     + jnp.dot(h, whh_ref[...], preferred_element_type=_f32)
                     + bl_ref[...])
            g_i = gates[:, 0:_DP]
            g_f = gates[:, _DP:2 * _DP]
            g_g = gates[:, 2 * _DP:3 * _DP]
            g_o = gates[:, 3 * _DP:4 * _DP]
            c_new = (jax.nn.sigmoid(g_f) * c
                     + jax.nn.sigmoid(g_i) * jnp.tanh(g_g))
            h_new = jax.nn.sigmoid(g_o) * jnp.tanh(c_new)

            pre = encp_ref[t] + jnp.dot(h_new, wpred_ref[...],
                                        preferred_element_type=_f32)
            logits = (jnp.dot(jnp.tanh(pre), wout_ref[...],
                              preferred_element_type=_f32)
                      + bout_ref[...])

            m = jnp.max(logits, axis=1, keepdims=True)
            sym = jnp.argmax(logits, axis=1, keepdims=True).astype(jnp.int32)
            score = -jnp.log(jnp.sum(jnp.exp(logits - m),
                                     axis=1, keepdims=True))

            blank = jnp.logical_or(sym == _BLANK, t >= lens)
            h = jnp.where(blank, h, h_new)
            c = jnp.where(blank, c, c_new)
            lbl = jnp.where(blank, lbl, sym)
            emit = jnp.where(blank, _BLANK, sym)

            colmask = iota_c == tt
            labacc = jnp.where(colmask,
                               jnp.broadcast_to(emit, (_B, _TCH)), labacc)
            scacc = jnp.where(colmask,
                              jnp.broadcast_to(score, (_B, _TCH)), scacc)
            return h, c, lbl, labacc, scacc
        return body

    h = jnp.zeros((_B, _DP), _f32)
    c = jnp.zeros((_B, _DP), _f32)
    lbl = jnp.full((_B, 1), _BLANK, jnp.int32)
    for chunk in range(_T // _TCH):
        init = (h, c, lbl,
                jnp.zeros((_B, _TCH), jnp.int32),
                jnp.zeros((_B, _TCH), _f32))
        h, c, lbl, labacc, scacc = jax.lax.fori_loop(
            0, _TCH, step(chunk), init, unroll=8)
        lab_ref[:, chunk * _TCH:(chunk + 1) * _TCH] = labacc
        sc_ref[:, chunk * _TCH:(chunk + 1) * _TCH] = scacc


def _full(shape):
    return pl.BlockSpec(shape, lambda i: (0,) * len(shape))


@jax.jit
def kernel(encoded_outs, encoded_lens, embed, W_ih, W_hh, b_lstm,
           W_enc, W_pred, b_joint, W_out, b_out):
    enc_flat = jnp.transpose(encoded_outs, (1, 0, 2)).reshape(_T * _B, _DE)

    encp = pl.pallas_call(
        _proj_kernel,
        grid=(_T * _B // _MB,),
        in_specs=[
            pl.BlockSpec((_MB, _DE), lambda i: (i, 0)),
            pl.BlockSpec((_DE, _DJ), lambda i: (0, 0)),
            pl.BlockSpec((1, _DJ), lambda i: (0, 0)),
        ],
        out_specs=pl.BlockSpec((_MB, _DJ), lambda i: (i, 0)),
        out_shape=jax.ShapeDtypeStruct((_T * _B, _DJ), _f32),
    )(enc_flat, W_enc, b_joint[None, :])
    encp = encp.reshape(_T, _B, _DJ)

    lens_b = jnp.broadcast_to(encoded_lens.astype(jnp.int32)[:, None],
                              (_B, 128))

    labels, scores = pl.pallas_call(
        _decode_kernel,
        grid=(1,),
        in_specs=[
            _full((_T, _B, _DJ)),
            _full((_B, 128)),
            _full((_V, _DP)),
            _full((_DP, 4 * _DP)),
            _full((_DP, 4 * _DP)),
            _full((1, 4 * _DP)),
            _full((_DP, _DJ)),
            _full((_DJ, _V)),
            _full((1, _V)),
        ],
        out_specs=[
            _full((_B, _T)),
            _full((_B, _T)),
        ],
        out_shape=[
            jax.ShapeDtypeStruct((_B, _T), jnp.int32),
            jax.ShapeDtypeStruct((_B, _T), _f32),
        ],
        compiler_params=pltpu.CompilerParams(
            dimension_semantics=("arbitrary",)),
    )(encp, lens_b, embed, W_ih, W_hh, b_lstm[None, :],
      W_pred, W_out, b_out[None, :])
    return labels, scores


# speedup vs baseline: 1.2091x; 1.0204x over previous
import jax
import jax.numpy as jnp
from jax.experimental import pallas as pl
from jax.experimental.pallas import tpu as pltpu

_B = 16
_T = 512
_DE = 512
_DP = 320
_DJ = 320
_V = 1024
_BLANK = 0
_TCH = 128
_MB = 1024

_f32 = jnp.float32


def _proj_kernel(enc_ref, wenc_ref, bj_ref, out_ref):
    out_ref[...] = (jnp.dot(enc_ref[...], wenc_ref[...],
                            preferred_element_type=_f32)
                    + bj_ref[...])


def _decode_kernel(encp_ref, lens_ref, embed_ref,
                   wih_ref, whh_ref, bl_ref,
                   wpred_ref, wout_ref, bout_ref,
                   lab_ref, sc_ref):
    iota_v = jax.lax.broadcasted_iota(jnp.int32, (_B, _V), 1)
    iota_c = jax.lax.broadcasted_iota(jnp.int32, (_B, _TCH), 1)
    lens = lens_ref[...][:, :1]

    def step(chunk):
        def body(tt, carry):
            h, c, lbl, labacc, scacc = carry
            t = chunk * _TCH + tt

            onehot = (iota_v == lbl).astype(_f32)
            emb = jnp.dot(onehot, embed_ref[...],
                          preferred_element_type=_f32)

            gates = (jnp.dot(emb, wih_ref[...], preferred_element_type=_f32)
                     + jnp.dot(h, whh_ref[...], preferred_element_type=_f32)
                     + bl_ref[...])
            g_i = gates[:, 0:_DP]
            g_f = gates[:, _DP:2 * _DP]
            g_g = gates[:, 2 * _DP:3 * _DP]
            g_o = gates[:, 3 * _DP:4 * _DP]
            c_new = (jax.nn.sigmoid(g_f) * c
                     + jax.nn.sigmoid(g_i) * jnp.tanh(g_g))
            h_new = jax.nn.sigmoid(g_o) * jnp.tanh(c_new)

            pre = encp_ref[t] + jnp.dot(h_new, wpred_ref[...],
                                        preferred_element_type=_f32)
            jt = jnp.tanh(pre)
            la = (jnp.dot(jt, wout_ref[:, :_V // 2],
                          preferred_element_type=_f32)
                  + bout_ref[:, :_V // 2])
            lb = (jnp.dot(jt, wout_ref[:, _V // 2:],
                          preferred_element_type=_f32)
                  + bout_ref[:, _V // 2:])
            ma = jnp.max(la, axis=1, keepdims=True)
            ia = jnp.argmax(la, axis=1, keepdims=True).astype(jnp.int32)
            mb = jnp.max(lb, axis=1, keepdims=True)
            ib = (jnp.argmax(lb, axis=1, keepdims=True).astype(jnp.int32)
                  + _V // 2)
            bwins = mb > ma
            m = jnp.where(bwins, mb, ma)
            sym = jnp.where(bwins, ib, ia)
            score = -jnp.log(jnp.sum(jnp.exp(la - m), axis=1, keepdims=True)
                             + jnp.sum(jnp.exp(lb - m), axis=1,
                                       keepdims=True))

            blank = jnp.logical_or(sym == _BLANK, t >= lens)
            h = jnp.where(blank, h, h_new)
            c = jnp.where(blank, c, c_new)
            lbl = jnp.where(blank, lbl, sym)
            emit = jnp.where(blank, _BLANK, sym)

            colmask = iota_c == tt
            labacc = jnp.where(colmask,
                               jnp.broadcast_to(emit, (_B, _TCH)), labacc)
            scacc = jnp.where(colmask,
                              jnp.broadcast_to(score, (_B, _TCH)), scacc)
            return h, c, lbl, labacc, scacc
        return body

    h = jnp.zeros((_B, _DP), _f32)
    c = jnp.zeros((_B, _DP), _f32)
    lbl = jnp.full((_B, 1), _BLANK, jnp.int32)
    for chunk in range(_T // _TCH):
        init = (h, c, lbl,
                jnp.zeros((_B, _TCH), jnp.int32),
                jnp.zeros((_B, _TCH), _f32))
        h, c, lbl, labacc, scacc = jax.lax.fori_loop(
            0, _TCH, step(chunk), init, unroll=8)
        lab_ref[:, chunk * _TCH:(chunk + 1) * _TCH] = labacc
        sc_ref[:, chunk * _TCH:(chunk + 1) * _TCH] = scacc


def _full(shape):
    return pl.BlockSpec(shape, lambda i: (0,) * len(shape))


@jax.jit
def kernel(encoded_outs, encoded_lens, embed, W_ih, W_hh, b_lstm,
           W_enc, W_pred, b_joint, W_out, b_out):
    enc_flat = jnp.transpose(encoded_outs, (1, 0, 2)).reshape(_T * _B, _DE)

    encp = pl.pallas_call(
        _proj_kernel,
        grid=(_T * _B // _MB,),
        in_specs=[
            pl.BlockSpec((_MB, _DE), lambda i: (i, 0)),
            pl.BlockSpec((_DE, _DJ), lambda i: (0, 0)),
            pl.BlockSpec((1, _DJ), lambda i: (0, 0)),
        ],
        out_specs=pl.BlockSpec((_MB, _DJ), lambda i: (i, 0)),
        out_shape=jax.ShapeDtypeStruct((_T * _B, _DJ), _f32),
    )(enc_flat, W_enc, b_joint[None, :])
    encp = encp.reshape(_T, _B, _DJ)

    lens_b = jnp.broadcast_to(encoded_lens.astype(jnp.int32)[:, None],
                              (_B, 128))

    labels, scores = pl.pallas_call(
        _decode_kernel,
        grid=(1,),
        in_specs=[
            _full((_T, _B, _DJ)),
            _full((_B, 128)),
            _full((_V, _DP)),
            _full((_DP, 4 * _DP)),
            _full((_DP, 4 * _DP)),
            _full((1, 4 * _DP)),
            _full((_DP, _DJ)),
            _full((_DJ, _V)),
            _full((1, _V)),
        ],
        out_specs=[
            _full((_B, _T)),
            _full((_B, _T)),
        ],
        out_shape=[
            jax.ShapeDtypeStruct((_B, _T), jnp.int32),
            jax.ShapeDtypeStruct((_B, _T), _f32),
        ],
        compiler_params=pltpu.CompilerParams(
            dimension_semantics=("arbitrary",)),
    )(encp, lens_b, embed, W_ih, W_hh, b_lstm[None, :],
      W_pred, W_out, b_out[None, :])
    return labels, scores
